# Initial kernel scaffold; baseline (speedup 1.0000x reference)
import jax
import jax.numpy as jnp
from jax.experimental import pallas as pl


def kernel(seq, seq_len, edge_index, graph_ids, emb, Wih_f, Whh_f, b_f, Wih_b, Whh_b, b_b, W1, b1, W2, b2, W3, b3, Wr, br):
    raise NotImplementedError("write your pallas kernel here")


# speedup vs baseline: 8.0462x; 8.0462x over previous
import functools

import jax
import jax.numpy as jnp
from jax import lax
from jax.experimental import pallas as pl
from jax.experimental.pallas import tpu as pltpu
from jax.experimental.pallas import tpu_sc as plsc

N_NODES = 10000
NP = 10240
N_EDGES = 320000
NGRAPH = 128
B = 128
T = 200
H = 64
F = 128

NC, NS = 2, 16
NW = NC * NS
EPW = N_EDGES // NW
EBLK = 128
NFULL = EPW // EBLK
ETAIL = EPW - NFULL * EBLK
RPT = NP // NW
RPS = NP // NS

_f32 = jnp.float32
_i32 = jnp.int32


def _sc_mesh():
  return plsc.VectorSubcoreMesh(
      core_axis_name="c", subcore_axis_name="s", num_cores=NC, num_subcores=NS)


def _sc_deg(dst):
  def body(dst_hbm, out_hbm, idx_v, idx16_v, ones_v, ones16_v, buf_v, acc):
    c = lax.axis_index("c")
    s = lax.axis_index("s")
    wid = s * NC + c

    def zloop(i, _):
      buf_v[pl.ds(i * 16, 16)] = jnp.zeros((16,), _f32)
      return 0
    lax.fori_loop(0, RPS // 16, zloop, 0)
    for k in range(EBLK // 16):
      ones_v[pl.ds(k * 16, 16)] = jnp.ones((16,), _f32)
    ones16_v[...] = jnp.ones((16,), _f32)
    pltpu.sync_copy(buf_v, acc.at[pl.ds(s * RPS, RPS)])
    plsc.subcore_barrier()

    base = wid * EPW

    def blk(j, _):
      pltpu.sync_copy(dst_hbm.at[pl.ds(base + j * EBLK, EBLK)], idx_v)
      pltpu.sync_copy(ones_v, acc.at[idx_v], add=True)
      return 0
    lax.fori_loop(0, NFULL, blk, 0)
    pltpu.sync_copy(dst_hbm.at[pl.ds(base + NFULL * EBLK, ETAIL)], idx16_v)
    pltpu.sync_copy(ones16_v, acc.at[idx16_v], add=True)

    plsc.subcore_barrier()
    pltpu.sync_copy(acc.at[pl.ds(s * RPS, RPS)], buf_v)
    pltpu.sync_copy(buf_v, out_hbm.at[pl.ds(c * NP + s * RPS, RPS)])

  return pl.kernel(
      body,
      out_type=jax.ShapeDtypeStruct((NC * NP,), _f32),
      mesh=_sc_mesh(),
      scratch_types=[
          pltpu.VMEM((EBLK,), _i32),
          pltpu.VMEM((ETAIL,), _i32),
          pltpu.VMEM((EBLK,), _f32),
          pltpu.VMEM((ETAIL,), _f32),
          pltpu.VMEM((RPS,), _f32),
          pltpu.VMEM_SHARED((NP,), _f32),
      ],
  )(dst)


def _sc_agg1(src, dst, deg):
  def body(src_hbm, dst_hbm, deg_hbm, out_hbm,
           sidx_v, didx_v, s16_v, d16_v, vals_v, v16_v, buf_v, acc, sem):
    c = lax.axis_index("c")
    s = lax.axis_index("s")
    wid = s * NC + c

    def zloop(i, _):
      buf_v[pl.ds(i * 16, 16)] = jnp.zeros((16,), _f32)
      return 0
    lax.fori_loop(0, RPS // 16, zloop, 0)
    pltpu.sync_copy(buf_v, acc.at[pl.ds(s * RPS, RPS)])
    plsc.subcore_barrier()

    base = wid * EPW

    def blk(j, _):
      pltpu.sync_copy(src_hbm.at[pl.ds(base + j * EBLK, EBLK)], sidx_v)
      pltpu.sync_copy(dst_hbm.at[pl.ds(base + j * EBLK, EBLK)], didx_v)
      pltpu.async_copy(deg_hbm.at[sidx_v], vals_v, sem).wait()
      pltpu.sync_copy(vals_v, acc.at[didx_v], add=True)
      return 0
    lax.fori_loop(0, NFULL, blk, 0)
    pltpu.sync_copy(src_hbm.at[pl.ds(base + NFULL * EBLK, ETAIL)], s16_v)
    pltpu.sync_copy(dst_hbm.at[pl.ds(base + NFULL * EBLK, ETAIL)], d16_v)
    pltpu.async_copy(deg_hbm.at[s16_v], v16_v, sem).wait()
    pltpu.sync_copy(v16_v, acc.at[d16_v], add=True)

    plsc.subcore_barrier()
    pltpu.sync_copy(acc.at[pl.ds(s * RPS, RPS)], buf_v)
    pltpu.sync_copy(buf_v, out_hbm.at[pl.ds(c * NP + s * RPS, RPS)])

  return pl.kernel(
      body,
      out_type=jax.ShapeDtypeStruct((NC * NP,), _f32),
      mesh=_sc_mesh(),
      scratch_types=[
          pltpu.VMEM((EBLK,), _i32),
          pltpu.VMEM((EBLK,), _i32),
          pltpu.VMEM((ETAIL,), _i32),
          pltpu.VMEM((ETAIL,), _i32),
          pltpu.VMEM((EBLK,), _f32),
          pltpu.VMEM((ETAIL,), _f32),
          pltpu.VMEM((RPS,), _f32),
          pltpu.VMEM_SHARED((NP,), _f32),
          pltpu.SemaphoreType.DMA,
      ],
  )(src, dst, deg)


def _sc_aggF(src, dst, table):
  def body(src_hbm, dst_hbm, tab_hbm, out_hbm,
           sidx_v, didx_v, s16_v, d16_v, rows_v, rows16_v, acc, sem):
    c = lax.axis_index("c")
    s = lax.axis_index("s")
    wid = s * NC + c

    def zr(i, _):
      for k in range(F // 16):
        rows_v[i, pl.ds(k * 16, 16)] = jnp.zeros((16,), _f32)
      return 0
    lax.fori_loop(0, EBLK, zr, 0)
    for j in range(RPS // EBLK):
      pltpu.sync_copy(rows_v, acc.at[pl.ds(s * RPS + j * EBLK, EBLK)])
    plsc.subcore_barrier()

    base = wid * EPW

    def blk(j, _):
      pltpu.sync_copy(src_hbm.at[pl.ds(base + j * EBLK, EBLK)], sidx_v)
      pltpu.sync_copy(dst_hbm.at[pl.ds(base + j * EBLK, EBLK)], didx_v)
      pltpu.async_copy(tab_hbm.at[sidx_v], rows_v, sem).wait()
      pltpu.sync_copy(rows_v, acc.at[didx_v], add=True)
      return 0
    lax.fori_loop(0, NFULL, blk, 0)
    pltpu.sync_copy(src_hbm.at[pl.ds(base + NFULL * EBLK, ETAIL)], s16_v)
    pltpu.sync_copy(dst_hbm.at[pl.ds(base + NFULL * EBLK, ETAIL)], d16_v)
    pltpu.async_copy(tab_hbm.at[s16_v], rows16_v, sem).wait()
    pltpu.sync_copy(rows16_v, acc.at[d16_v], add=True)

    plsc.subcore_barrier()
    for j in range(RPS // EBLK):
      pltpu.sync_copy(acc.at[pl.ds(s * RPS + j * EBLK, EBLK)], rows_v)
      pltpu.sync_copy(rows_v, out_hbm.at[pl.ds(c * NP + s * RPS + j * EBLK, EBLK)])

  return pl.kernel(
      body,
      out_type=jax.ShapeDtypeStruct((NC * NP, F), _f32),
      mesh=_sc_mesh(),
      scratch_types=[
          pltpu.VMEM((EBLK,), _i32),
          pltpu.VMEM((EBLK,), _i32),
          pltpu.VMEM((ETAIL,), _i32),
          pltpu.VMEM((ETAIL,), _i32),
          pltpu.VMEM((EBLK, F), _f32),
          pltpu.VMEM((ETAIL, F), _f32),
          pltpu.VMEM_SHARED((NP, F), _f32),
          pltpu.SemaphoreType.DMA,
      ],
  )(src, dst, table)


def _lstm_body(seq_ref, len_ref, emb_ref, wihf_ref, whhf_ref, bf_ref,
               wihb_ref, whhb_ref, bb_ref, out_ref):
  ef = lax.dot_general(emb_ref[...], wihf_ref[...], (((1,), (1,)), ((), ())),
                       preferred_element_type=_f32)
  eb = lax.dot_general(emb_ref[...], wihb_ref[...], (((1,), (1,)), ((), ())),
                       preferred_element_type=_f32)
  whhf = whhf_ref[...]
  whhb = whhb_ref[...]
  bf = bf_ref[...]
  bb = bb_ref[...]
  lens = len_ref[...]

  def gates(oh_t, h, e, whh, b):
    g = (lax.dot_general(oh_t, e, (((0,), (0,)), ((), ())),
                         preferred_element_type=_f32)
         + lax.dot_general(h, whh, (((1,), (1,)), ((), ())),
                           preferred_element_type=_f32) + b)
    i = g[:, 0:H]
    f = g[:, H:2 * H]
    gg = g[:, 2 * H:3 * H]
    o = g[:, 3 * H:4 * H]
    return i, f, gg, o

  def onehot_t(t):
    row = seq_ref[t, :]
    return (lax.broadcasted_iota(_i32, (21, B), 0) == row[None, :]).astype(_f32)

  def step(t, carry):
    hf, cf, hb, cb = carry
    i, f, gg, o = gates(onehot_t(t), hf, ef, whhf, bf)
    cn = jax.nn.sigmoid(f) * cf + jax.nn.sigmoid(i) * jnp.tanh(gg)
    hn = jax.nn.sigmoid(o) * jnp.tanh(cn)
    mf = t < lens
    hf = jnp.where(mf, hn, hf)
    cf = jnp.where(mf, cn, cf)
    sp = (T - 1) - t
    i, f, gg, o = gates(onehot_t(sp), hb, eb, whhb, bb)
    cn = jax.nn.sigmoid(f) * cb + jax.nn.sigmoid(i) * jnp.tanh(gg)
    hn = jax.nn.sigmoid(o) * jnp.tanh(cn)
    mb = sp < lens
    hb = jnp.where(mb, hn, hb)
    cb = jnp.where(mb, cn, cb)
    return hf, cf, hb, cb

  z = jnp.zeros((B, H), _f32)
  hf, _, hb, _ = lax.fori_loop(0, T, step, (z, z, z, z))
  out_ref[:, 0:H] = hf
  out_ref[:, H:2 * H] = hb


def _lstm(seq_tm, lens_col, emb, wihf, whhf, bf, wihb, whhb, bb):
  return pl.pallas_call(
      _lstm_body,
      out_shape=jax.ShapeDtypeStruct((B, 2 * H), _f32),
  )(seq_tm, lens_col, emb, wihf, whhf, bf, wihb, whhb, bb)


def _degsum_body(d0, d1, out):
  out[...] = d0[...] + d1[...]


def _degsum(d0, d1):
  nblk = 2048
  col = pl.BlockSpec((nblk, 1), lambda i: (i, 0))
  return pl.pallas_call(
      _degsum_body,
      grid=(NP // nblk,),
      in_specs=[col, col],
      out_specs=col,
      out_shape=jax.ShapeDtypeStruct((NP, 1), _f32),
  )(d0, d1)


def _h1_body(deg, q0, q1, w1, b1, out):
  d = deg[...]
  q = q0[...] + q1[...]
  a = jnp.where(d > 0, q / jnp.maximum(d, 1.0), d)
  out[...] = jnp.maximum(a * w1[...] + b1[...], 0.0)


def _h1(deg, q0, q1, w1row, b1row):
  nblk = 2048
  grid = NP // nblk
  col = pl.BlockSpec((nblk, 1), lambda i: (i, 0))
  return pl.pallas_call(
      _h1_body,
      grid=(grid,),
      in_specs=[col, col, col,
                pl.BlockSpec((1, F), lambda i: (0, 0)),
                pl.BlockSpec((1, F), lambda i: (0, 0))],
      out_specs=pl.BlockSpec((nblk, F), lambda i: (i, 0)),
      out_shape=jax.ShapeDtypeStruct((NP, F), _f32),
  )(deg, q0, q1, w1row, b1row)


def _h2g3_body(p0, p1, h1, deg, w2, b2, w3, out):
  d = deg[...]
  a2 = jnp.where(d > 0, (p0[...] + p1[...]) / jnp.maximum(d, 1.0), h1[...])
  h2 = jnp.maximum(
      lax.dot_general(a2, w2[...], (((1,), (1,)), ((), ())),
                      preferred_element_type=_f32) + b2[...], 0.0)
  out[...] = lax.dot_general(h2, w3[...], (((1,), (1,)), ((), ())),
                             preferred_element_type=_f32)


def _h2g3(p0, p1, h1, deg, w2, b2, w3):
  nblk = 2048
  grid = NP // nblk
  row = pl.BlockSpec((nblk, F), lambda i: (i, 0))
  col = pl.BlockSpec((nblk, 1), lambda i: (i, 0))
  return pl.pallas_call(
      _h2g3_body,
      grid=(grid,),
      in_specs=[row, row, row, col,
                pl.BlockSpec((256, F), lambda i: (0, 0)),
                pl.BlockSpec((1, 256), lambda i: (0, 0)),
                pl.BlockSpec((F, 256), lambda i: (0, 0))],
      out_specs=row,
      out_shape=jax.ShapeDtypeStruct((NP, F), _f32),
  )(p0, p1, h1, deg, w2, b2, w3)


def _final_body(p0, p1, g3, deg, gid, b3, co, wrl, wrg, br, out,
                gsum_sc, gcnt_sc):
  i = pl.program_id(0)
  nblk = p0.shape[0]
  d = deg[...]
  a3 = jnp.where(d > 0, (p0[...] + p1[...]) / jnp.maximum(d, 1.0), g3[...])
  h3 = jnp.maximum(a3 + b3[...], 0.0)
  oh = (gid[...] == lax.broadcasted_iota(_i32, (nblk, NGRAPH), 1)).astype(_f32)
  part = lax.dot_general(oh, h3, (((0,), (0,)), ((), ())),
                         preferred_element_type=_f32)
  cnt = lax.dot_general(oh, jnp.ones((nblk, 1), _f32), (((0,), (0,)), ((), ())),
                        preferred_element_type=_f32)

  @pl.when(i == 0)
  def _():
    gsum_sc[...] = jnp.zeros_like(gsum_sc)
    gcnt_sc[...] = jnp.zeros_like(gcnt_sc)

  gsum_sc[...] += part
  gcnt_sc[...] += cnt

  @pl.when(i == pl.num_programs(0) - 1)
  def _():
    gmean = gsum_sc[...] / jnp.maximum(gcnt_sc[...], 1.0)
    out[...] = (lax.dot_general(co[...], wrl[...], (((1,), (1,)), ((), ())),
                                preferred_element_type=_f32)
                + lax.dot_general(gmean, wrg[...], (((1,), (1,)), ((), ())),
                                  preferred_element_type=_f32)
                + br[...])


def _final(p0, p1, g3, deg, gid_col, b3row, co, wrl, wrg, br):
  nblk = 2000
  grid = N_NODES // nblk
  row = pl.BlockSpec((nblk, F), lambda i: (i, 0))
  col = pl.BlockSpec((nblk, 1), lambda i: (i, 0))
  fixed = lambda shape: pl.BlockSpec(shape, lambda i: (0, 0))
  return pl.pallas_call(
      _final_body,
      grid=(grid,),
      in_specs=[row, row, row, col, col,
                fixed((1, F)), fixed((B, 2 * H)), fixed((1, 2 * H)),
                fixed((1, F)), fixed((1, 1))],
      out_specs=fixed((B, 1)),
      out_shape=jax.ShapeDtypeStruct((B, 1), _f32),
      scratch_shapes=[pltpu.VMEM((NGRAPH, F), _f32),
                      pltpu.VMEM((NGRAPH, 1), _f32)],
  )(p0, p1, g3, deg, gid_col, b3row, co, wrl, wrg, br)


def kernel(seq, seq_len, edge_index, graph_ids, emb, Wih_f, Whh_f, b_f,
           Wih_b, Whh_b, b_b, W1, b1, W2, b2, W3, b3, Wr, br):
  src = edge_index[0]
  dst = edge_index[1]

  seq_tm = seq.astype(_i32).T
  lens_col = seq_len.astype(_i32).reshape(B, 1)
  co = _lstm(seq_tm, lens_col, emb.astype(_f32), Wih_f, Whh_f,
             b_f.reshape(1, 4 * H), Wih_b, Whh_b, b_b.reshape(1, 4 * H))

  dpart = _sc_deg(dst)
  deg_col = _degsum(dpart[:NP].reshape(NP, 1), dpart[NP:].reshape(NP, 1))
  qpart = _sc_agg1(src, dst, deg_col.reshape(NP))
  q0 = qpart[:NP].reshape(NP, 1)
  q1 = qpart[NP:].reshape(NP, 1)
  h1 = _h1(deg_col, q0, q1, W1.reshape(1, F), b1.reshape(1, F))
  p2 = _sc_aggF(src, dst, h1)
  g3 = _h2g3(p2[:NP], p2[NP:], h1, deg_col, W2, b2.reshape(1, 256), W3)
  p3 = _sc_aggF(src, dst, g3)

  gid_col = graph_ids.astype(_i32).reshape(N_NODES, 1)
  out = _final(p3[:NP], p3[NP:], g3, deg_col, gid_col, b3.reshape(1, F),
               co, Wr[:, :2 * H], Wr[:, 2 * H:], br.reshape(1, 1))
  return out


# speedup vs baseline: 13.1930x; 1.6397x over previous
import functools

import jax
import jax.numpy as jnp
from jax import lax
from jax.experimental import pallas as pl
from jax.experimental.pallas import tpu as pltpu
from jax.experimental.pallas import tpu_sc as plsc

N_NODES = 10000
NP = 10240
N_EDGES = 320000
NGRAPH = 128
B = 128
T = 200
H = 64
F = 128

NC, NS = 2, 16
NW = NC * NS
EPW = N_EDGES // NW
EBLK = 128
NFULL = EPW // EBLK
ETAIL = EPW - NFULL * EBLK
RPT = NP // NW
RPS = NP // NS

_f32 = jnp.float32
_i32 = jnp.int32


def _sc_mesh():
  return plsc.VectorSubcoreMesh(
      core_axis_name="c", subcore_axis_name="s", num_cores=NC, num_subcores=NS)


def _sc_deg(dst):
  def body(dst_hbm, out_hbm, idx_v, idx16_v, ones_v, ones16_v, buf_v, acc):
    c = lax.axis_index("c")
    s = lax.axis_index("s")
    wid = s * NC + c

    def zloop(i, _):
      buf_v[pl.ds(i * 16, 16)] = jnp.zeros((16,), _f32)
      return 0
    lax.fori_loop(0, RPS // 16, zloop, 0)
    for k in range(EBLK // 16):
      ones_v[pl.ds(k * 16, 16)] = jnp.ones((16,), _f32)
    ones16_v[...] = jnp.ones((16,), _f32)
    pltpu.sync_copy(buf_v, acc.at[pl.ds(s * RPS, RPS)])
    plsc.subcore_barrier()

    base = wid * EPW

    def blk(j, _):
      pltpu.sync_copy(dst_hbm.at[pl.ds(base + j * EBLK, EBLK)], idx_v)
      pltpu.sync_copy(ones_v, acc.at[idx_v], add=True)
      return 0
    lax.fori_loop(0, NFULL, blk, 0)
    pltpu.sync_copy(dst_hbm.at[pl.ds(base + NFULL * EBLK, ETAIL)], idx16_v)
    pltpu.sync_copy(ones16_v, acc.at[idx16_v], add=True)

    plsc.subcore_barrier()
    pltpu.sync_copy(acc.at[pl.ds(s * RPS, RPS)], buf_v)
    pltpu.sync_copy(buf_v, out_hbm.at[pl.ds(c * NP + s * RPS, RPS)])

  return pl.kernel(
      body,
      out_type=jax.ShapeDtypeStruct((NC * NP,), _f32),
      mesh=_sc_mesh(),
      scratch_types=[
          pltpu.VMEM((EBLK,), _i32),
          pltpu.VMEM((ETAIL,), _i32),
          pltpu.VMEM((EBLK,), _f32),
          pltpu.VMEM((ETAIL,), _f32),
          pltpu.VMEM((RPS,), _f32),
          pltpu.VMEM_SHARED((NP,), _f32),
      ],
  )(dst)


def _sc_agg1(src, dst, deg):
  def body(src_hbm, dst_hbm, deg_hbm, out_hbm,
           sidx, didx, s16_v, d16_v, vals, v16_v, buf_v, acc, gsem, ssem, isem):
    c = lax.axis_index("c")
    s = lax.axis_index("s")
    wid = s * NC + c

    def zloop(i, _):
      buf_v[pl.ds(i * 16, 16)] = jnp.zeros((16,), _f32)
      return 0
    lax.fori_loop(0, RPS // 16, zloop, 0)
    pltpu.sync_copy(buf_v, acc.at[pl.ds(s * RPS, RPS)])
    plsc.subcore_barrier()

    base = wid * EPW

    def start_idx(g):
      gi = lax.rem(g, 4)
      pltpu.async_copy(src_hbm.at[pl.ds(base + g * EBLK, EBLK)], sidx.at[gi], isem.at[gi])
      pltpu.async_copy(dst_hbm.at[pl.ds(base + g * EBLK, EBLK)], didx.at[gi], isem.at[gi])

    start_idx(0)
    start_idx(1)

    def it(g, _):
      si = lax.rem(g, 4)
      sr = lax.rem(g, 2)

      @pl.when(g >= 2)
      def _():
        pltpu.make_async_copy(deg_hbm.at[pl.ds(0, EBLK)], vals.at[sr], ssem.at[sr]).wait()

      @pl.when(g + 2 < NFULL)
      def _():
        start_idx(g + 2)

      pltpu.make_async_copy(src_hbm.at[pl.ds(base, EBLK)], sidx.at[si], isem.at[si]).wait()
      pltpu.make_async_copy(src_hbm.at[pl.ds(base, EBLK)], didx.at[si], isem.at[si]).wait()
      pltpu.async_copy(deg_hbm.at[sidx.at[si]], vals.at[sr], gsem).wait()
      pltpu.async_copy(vals.at[sr], acc.at[didx.at[si]], ssem.at[sr], add=True)
      return 0
    lax.fori_loop(0, NFULL, it, 0)
    pltpu.make_async_copy(deg_hbm.at[pl.ds(0, EBLK)], vals.at[0], ssem.at[0]).wait()
    pltpu.make_async_copy(deg_hbm.at[pl.ds(0, EBLK)], vals.at[1], ssem.at[1]).wait()

    pltpu.sync_copy(src_hbm.at[pl.ds(base + NFULL * EBLK, ETAIL)], s16_v)
    pltpu.sync_copy(dst_hbm.at[pl.ds(base + NFULL * EBLK, ETAIL)], d16_v)
    pltpu.async_copy(deg_hbm.at[s16_v], v16_v, gsem).wait()
    pltpu.sync_copy(v16_v, acc.at[d16_v], add=True)

    plsc.subcore_barrier()
    pltpu.sync_copy(acc.at[pl.ds(s * RPS, RPS)], buf_v)
    pltpu.sync_copy(buf_v, out_hbm.at[pl.ds(c * NP + s * RPS, RPS)])

  return pl.kernel(
      body,
      out_type=jax.ShapeDtypeStruct((NC * NP,), _f32),
      mesh=_sc_mesh(),
      scratch_types=[
          pltpu.VMEM((4, EBLK), _i32),
          pltpu.VMEM((4, EBLK), _i32),
          pltpu.VMEM((ETAIL,), _i32),
          pltpu.VMEM((ETAIL,), _i32),
          pltpu.VMEM((2, EBLK), _f32),
          pltpu.VMEM((ETAIL,), _f32),
          pltpu.VMEM((RPS,), _f32),
          pltpu.VMEM_SHARED((NP,), _f32),
          pltpu.SemaphoreType.DMA,
          pltpu.SemaphoreType.DMA((2,)),
          pltpu.SemaphoreType.DMA((4,)),
      ],
  )(src, dst, deg)


def _sc_aggF(src, dst, table):
  def body(src_hbm, dst_hbm, tab_hbm, out_hbm,
           sidx, didx, s16_v, d16_v, rows, rows16_v, acc, gsem, ssem, isem):
    c = lax.axis_index("c")
    s = lax.axis_index("s")
    wid = s * NC + c

    def zr(i, _):
      for k in range(F // 16):
        rows[0, i, pl.ds(k * 16, 16)] = jnp.zeros((16,), _f32)
      return 0
    lax.fori_loop(0, EBLK, zr, 0)
    for j in range(RPS // EBLK):
      pltpu.sync_copy(rows.at[0], acc.at[pl.ds(s * RPS + j * EBLK, EBLK)])
    plsc.subcore_barrier()

    base = wid * EPW

    def start_idx(g):
      gi = lax.rem(g, 4)
      pltpu.async_copy(src_hbm.at[pl.ds(base + g * EBLK, EBLK)], sidx.at[gi], isem.at[gi])
      pltpu.async_copy(dst_hbm.at[pl.ds(base + g * EBLK, EBLK)], didx.at[gi], isem.at[gi])

    start_idx(0)
    start_idx(1)

    def it(g, _):
      si = lax.rem(g, 4)
      sr = lax.rem(g, 2)

      @pl.when(g >= 2)
      def _():
        pltpu.make_async_copy(tab_hbm.at[pl.ds(0, EBLK)], rows.at[sr], ssem.at[sr]).wait()

      @pl.when(g + 2 < NFULL)
      def _():
        start_idx(g + 2)

      pltpu.make_async_copy(src_hbm.at[pl.ds(base, EBLK)], sidx.at[si], isem.at[si]).wait()
      pltpu.make_async_copy(src_hbm.at[pl.ds(base, EBLK)], didx.at[si], isem.at[si]).wait()
      pltpu.async_copy(tab_hbm.at[sidx.at[si]], rows.at[sr], gsem).wait()
      pltpu.async_copy(rows.at[sr], acc.at[didx.at[si]], ssem.at[sr], add=True)
      return 0
    lax.fori_loop(0, NFULL, it, 0)
    pltpu.make_async_copy(tab_hbm.at[pl.ds(0, EBLK)], rows.at[0], ssem.at[0]).wait()
    pltpu.make_async_copy(tab_hbm.at[pl.ds(0, EBLK)], rows.at[1], ssem.at[1]).wait()

    pltpu.sync_copy(src_hbm.at[pl.ds(base + NFULL * EBLK, ETAIL)], s16_v)
    pltpu.sync_copy(dst_hbm.at[pl.ds(base + NFULL * EBLK, ETAIL)], d16_v)
    pltpu.async_copy(tab_hbm.at[s16_v], rows16_v, gsem).wait()
    pltpu.sync_copy(rows16_v, acc.at[d16_v], add=True)

    plsc.subcore_barrier()
    for j in range(RPS // EBLK):
      pltpu.sync_copy(acc.at[pl.ds(s * RPS + j * EBLK, EBLK)], rows.at[0])
      pltpu.sync_copy(rows.at[0], out_hbm.at[pl.ds(c * NP + s * RPS + j * EBLK, EBLK)])

  return pl.kernel(
      body,
      out_type=jax.ShapeDtypeStruct((NC * NP, F), _f32),
      mesh=_sc_mesh(),
      scratch_types=[
          pltpu.VMEM((4, EBLK), _i32),
          pltpu.VMEM((4, EBLK), _i32),
          pltpu.VMEM((ETAIL,), _i32),
          pltpu.VMEM((ETAIL,), _i32),
          pltpu.VMEM((2, EBLK, F), _f32),
          pltpu.VMEM((ETAIL, F), _f32),
          pltpu.VMEM_SHARED((NP, F), _f32),
          pltpu.SemaphoreType.DMA,
          pltpu.SemaphoreType.DMA((2,)),
          pltpu.SemaphoreType.DMA((4,)),
      ],
  )(src, dst, table)


def _lstm_body(seq_ref, len_ref, emb_ref, wihf_ref, whhf_ref, bf_ref,
               wihb_ref, whhb_ref, bb_ref, out_ref):
  ef = lax.dot_general(emb_ref[...], wihf_ref[...], (((1,), (1,)), ((), ())),
                       preferred_element_type=_f32)
  eb = lax.dot_general(emb_ref[...], wihb_ref[...], (((1,), (1,)), ((), ())),
                       preferred_element_type=_f32)
  z21 = jnp.zeros((21, 4 * H), _f32)
  ecat = jnp.concatenate(
      [jnp.concatenate([ef, z21], 1), jnp.concatenate([z21, eb], 1)], 0)
  zw = jnp.zeros((4 * H, H), _f32)
  wcat = jnp.concatenate(
      [jnp.concatenate([whhf_ref[...], zw], 1),
       jnp.concatenate([zw, whhb_ref[...]], 1)], 0)
  bcat = jnp.concatenate([bf_ref[...], bb_ref[...]], 1)
  lens = len_ref[...]

  def onehot_t(t):
    row = seq_ref[t, :]
    return (lax.broadcasted_iota(_i32, (21, B), 0) == row[None, :]).astype(_f32)

  def step(t, carry):
    hf, cf, hb, cb = carry
    sp = (T - 1) - t
    a = jnp.concatenate([onehot_t(t), onehot_t(sp)], 0)
    hcat = jnp.concatenate([hf, hb], 1)
    g = (lax.dot_general(a, ecat, (((0,), (0,)), ((), ())),
                         preferred_element_type=_f32)
         + lax.dot_general(hcat, wcat, (((1,), (1,)), ((), ())),
                           preferred_element_type=_f32) + bcat)
    sg = jax.nn.sigmoid(g)
    th = jnp.tanh(g)
    cnf = sg[:, H:2 * H] * cf + sg[:, 0:H] * th[:, 2 * H:3 * H]
    cnb = sg[:, 5 * H:6 * H] * cb + sg[:, 4 * H:5 * H] * th[:, 6 * H:7 * H]
    tc2 = jnp.tanh(jnp.concatenate([cnf, cnb], 1))
    hnf = sg[:, 3 * H:4 * H] * tc2[:, 0:H]
    hnb = sg[:, 7 * H:8 * H] * tc2[:, H:2 * H]
    mf = t < lens
    mb = sp < lens
    hf = jnp.where(mf, hnf, hf)
    cf = jnp.where(mf, cnf, cf)
    hb = jnp.where(mb, hnb, hb)
    cb = jnp.where(mb, cnb, cb)
    return hf, cf, hb, cb

  z = jnp.zeros((B, H), _f32)
  hf, _, hb, _ = lax.fori_loop(0, T, step, (z, z, z, z))
  out_ref[:, 0:H] = hf
  out_ref[:, H:2 * H] = hb


def _lstm(seq_tm, lens_col, emb, wihf, whhf, bf, wihb, whhb, bb):
  return pl.pallas_call(
      _lstm_body,
      out_shape=jax.ShapeDtypeStruct((B, 2 * H), _f32),
  )(seq_tm, lens_col, emb, wihf, whhf, bf, wihb, whhb, bb)


def _degsum_body(d0, d1, out):
  out[...] = d0[...] + d1[...]


def _degsum(d0, d1):
  nblk = 2048
  col = pl.BlockSpec((nblk, 1), lambda i: (i, 0))
  return pl.pallas_call(
      _degsum_body,
      grid=(NP // nblk,),
      in_specs=[col, col],
      out_specs=col,
      out_shape=jax.ShapeDtypeStruct((NP, 1), _f32),
  )(d0, d1)


def _h1_body(deg, q0, q1, w1, b1, out):
  d = deg[...]
  q = q0[...] + q1[...]
  a = jnp.where(d > 0, q / jnp.maximum(d, 1.0), d)
  out[...] = jnp.maximum(a * w1[...] + b1[...], 0.0)


def _h1(deg, q0, q1, w1row, b1row):
  nblk = 2048
  grid = NP // nblk
  col = pl.BlockSpec((nblk, 1), lambda i: (i, 0))
  return pl.pallas_call(
      _h1_body,
      grid=(grid,),
      in_specs=[col, col, col,
                pl.BlockSpec((1, F), lambda i: (0, 0)),
                pl.BlockSpec((1, F), lambda i: (0, 0))],
      out_specs=pl.BlockSpec((nblk, F), lambda i: (i, 0)),
      out_shape=jax.ShapeDtypeStruct((NP, F), _f32),
  )(deg, q0, q1, w1row, b1row)


def _h2g3_body(p0, p1, h1, deg, w2, b2, w3, out):
  d = deg[...]
  a2 = jnp.where(d > 0, (p0[...] + p1[...]) / jnp.maximum(d, 1.0), h1[...])
  h2 = jnp.maximum(
      lax.dot_general(a2, w2[...], (((1,), (1,)), ((), ())),
                      preferred_element_type=_f32) + b2[...], 0.0)
  out[...] = lax.dot_general(h2, w3[...], (((1,), (1,)), ((), ())),
                             preferred_element_type=_f32)


def _h2g3(p0, p1, h1, deg, w2, b2, w3):
  nblk = 2048
  grid = NP // nblk
  row = pl.BlockSpec((nblk, F), lambda i: (i, 0))
  col = pl.BlockSpec((nblk, 1), lambda i: (i, 0))
  return pl.pallas_call(
      _h2g3_body,
      grid=(grid,),
      in_specs=[row, row, row, col,
                pl.BlockSpec((256, F), lambda i: (0, 0)),
                pl.BlockSpec((1, 256), lambda i: (0, 0)),
                pl.BlockSpec((F, 256), lambda i: (0, 0))],
      out_specs=row,
      out_shape=jax.ShapeDtypeStruct((NP, F), _f32),
  )(p0, p1, h1, deg, w2, b2, w3)


def _final_body(p0, p1, g3, deg, gid, b3, co, wrl, wrg, br, out,
                gsum_sc, gcnt_sc):
  i = pl.program_id(0)
  nblk = p0.shape[0]
  d = deg[...]
  a3 = jnp.where(d > 0, (p0[...] + p1[...]) / jnp.maximum(d, 1.0), g3[...])
  h3 = jnp.maximum(a3 + b3[...], 0.0)
  oh = (gid[...] == lax.broadcasted_iota(_i32, (nblk, NGRAPH), 1)).astype(_f32)
  part = lax.dot_general(oh, h3, (((0,), (0,)), ((), ())),
                         preferred_element_type=_f32)
  cnt = lax.dot_general(oh, jnp.ones((nblk, 1), _f32), (((0,), (0,)), ((), ())),
                        preferred_element_type=_f32)

  @pl.when(i == 0)
  def _():
    gsum_sc[...] = jnp.zeros_like(gsum_sc)
    gcnt_sc[...] = jnp.zeros_like(gcnt_sc)

  gsum_sc[...] += part
  gcnt_sc[...] += cnt

  @pl.when(i == pl.num_programs(0) - 1)
  def _():
    gmean = gsum_sc[...] / jnp.maximum(gcnt_sc[...], 1.0)
    out[...] = (lax.dot_general(co[...], wrl[...], (((1,), (1,)), ((), ())),
                                preferred_element_type=_f32)
                + lax.dot_general(gmean, wrg[...], (((1,), (1,)), ((), ())),
                                  preferred_element_type=_f32)
                + br[...])


def _final(p0, p1, g3, deg, gid_col, b3row, co, wrl, wrg, br):
  nblk = 2000
  grid = N_NODES // nblk
  row = pl.BlockSpec((nblk, F), lambda i: (i, 0))
  col = pl.BlockSpec((nblk, 1), lambda i: (i, 0))
  fixed = lambda shape: pl.BlockSpec(shape, lambda i: (0, 0))
  return pl.pallas_call(
      _final_body,
      grid=(grid,),
      in_specs=[row, row, row, col, col,
                fixed((1, F)), fixed((B, 2 * H)), fixed((1, 2 * H)),
                fixed((1, F)), fixed((1, 1))],
      out_specs=fixed((B, 1)),
      out_shape=jax.ShapeDtypeStruct((B, 1), _f32),
      scratch_shapes=[pltpu.VMEM((NGRAPH, F), _f32),
                      pltpu.VMEM((NGRAPH, 1), _f32)],
  )(p0, p1, g3, deg, gid_col, b3row, co, wrl, wrg, br)


def kernel(seq, seq_len, edge_index, graph_ids, emb, Wih_f, Whh_f, b_f,
           Wih_b, Whh_b, b_b, W1, b1, W2, b2, W3, b3, Wr, br):
  src = edge_index[0]
  dst = edge_index[1]

  seq_tm = seq.astype(_i32).T
  lens_col = seq_len.astype(_i32).reshape(B, 1)
  co = _lstm(seq_tm, lens_col, emb.astype(_f32), Wih_f, Whh_f,
             b_f.reshape(1, 4 * H), Wih_b, Whh_b, b_b.reshape(1, 4 * H))

  dpart = _sc_deg(dst)
  deg_col = _degsum(dpart[:NP].reshape(NP, 1), dpart[NP:].reshape(NP, 1))
  qpart = _sc_agg1(src, dst, deg_col.reshape(NP))
  q0 = qpart[:NP].reshape(NP, 1)
  q1 = qpart[NP:].reshape(NP, 1)
  h1 = _h1(deg_col, q0, q1, W1.reshape(1, F), b1.reshape(1, F))
  p2 = _sc_aggF(src, dst, h1)
  g3 = _h2g3(p2[:NP], p2[NP:], h1, deg_col, W2, b2.reshape(1, 256), W3)
  p3 = _sc_aggF(src, dst, g3)

  gid_col = graph_ids.astype(_i32).reshape(N_NODES, 1)
  out = _final(p3[:NP], p3[NP:], g3, deg_col, gid_col, b3.reshape(1, F),
               co, Wr[:, :2 * H], Wr[:, 2 * H:], br.reshape(1, 1))
  return out


# speedup vs baseline: 14.4005x; 1.0915x over previous
import jax
import jax.numpy as jnp
from jax import lax
from jax.experimental import pallas as pl
from jax.experimental.pallas import tpu as pltpu
from jax.experimental.pallas import tpu_sc as plsc

N_NODES = 10000
NP = 10240
N_EDGES = 320000
NGRAPH = 128
B = 128
T = 200
H = 64
F = 128
FD = 16

NC, NS = 2, 16
NW = NC * NS
EPW = N_EDGES // NW
EBLK = 128
NFULL = EPW // EBLK
ETAIL = EPW - NFULL * EBLK
RPS = NP // NS

_f32 = jnp.float32
_i32 = jnp.int32


def _sc_mesh():
  return plsc.VectorSubcoreMesh(
      core_axis_name="c", subcore_axis_name="s", num_cores=NC, num_subcores=NS)


def _sc_degrows(dst):
  def body(dst_hbm, out_hbm, didx, d16_v, rows, v16, acc, ssem, isem):
    c = lax.axis_index("c")
    s = lax.axis_index("s")
    wid = s * NC + c

    def zr(i, _):
      rows[0, i, pl.ds(0, 16)] = jnp.zeros((16,), _f32)
      return 0
    lax.fori_loop(0, EBLK, zr, 0)
    for j in range(RPS // EBLK):
      pltpu.sync_copy(rows.at[0], acc.at[pl.ds(s * RPS + j * EBLK, EBLK)])
    plsc.subcore_barrier()

    def ones_fill(i, _):
      rows[0, i, pl.ds(0, 16)] = jnp.ones((16,), _f32)
      rows[1, i, pl.ds(0, 16)] = jnp.ones((16,), _f32)
      return 0
    lax.fori_loop(0, EBLK, ones_fill, 0)
    for i in range(ETAIL):
      v16[i, pl.ds(0, 16)] = jnp.ones((16,), _f32)

    base = wid * EPW

    def start_idx(g):
      gi = lax.rem(g, 4)
      pltpu.async_copy(dst_hbm.at[pl.ds(base + g * EBLK, EBLK)], didx.at[gi],
                       isem.at[gi])

    start_idx(0)
    start_idx(1)

    def it(g, _):
      si = lax.rem(g, 4)
      sr = lax.rem(g, 2)

      @pl.when(g >= 2)
      def _():
        pltpu.make_async_copy(out_hbm.at[pl.ds(0, EBLK)], rows.at[sr],
                              ssem.at[sr]).wait()

      @pl.when(g + 2 < NFULL)
      def _():
        start_idx(g + 2)

      pltpu.make_async_copy(dst_hbm.at[pl.ds(base, EBLK)], didx.at[si],
                            isem.at[si]).wait()
      pltpu.async_copy(rows.at[sr], acc.at[didx.at[si]], ssem.at[sr], add=True)
      return 0
    lax.fori_loop(0, NFULL, it, 0)
    pltpu.make_async_copy(out_hbm.at[pl.ds(0, EBLK)], rows.at[0], ssem.at[0]).wait()
    pltpu.make_async_copy(out_hbm.at[pl.ds(0, EBLK)], rows.at[1], ssem.at[1]).wait()

    pltpu.sync_copy(dst_hbm.at[pl.ds(base + NFULL * EBLK, ETAIL)], d16_v)
    pltpu.sync_copy(v16, acc.at[d16_v], add=True)

    plsc.subcore_barrier()
    for j in range(RPS // EBLK):
      pltpu.sync_copy(acc.at[pl.ds(s * RPS + j * EBLK, EBLK)], rows.at[0])
      pltpu.sync_copy(rows.at[0], out_hbm.at[pl.ds(c * NP + s * RPS + j * EBLK, EBLK)])

  return pl.kernel(
      body,
      out_type=jax.ShapeDtypeStruct((NC * NP, FD), _f32),
      mesh=_sc_mesh(),
      compiler_params=pltpu.CompilerParams(use_tc_tiling_on_sc=False),
      scratch_types=[
          pltpu.VMEM((4, EBLK), _i32),
          pltpu.VMEM((ETAIL,), _i32),
          pltpu.VMEM((2, EBLK, FD), _f32),
          pltpu.VMEM((ETAIL, FD), _f32),
          pltpu.VMEM_SHARED((NP, FD), _f32),
          pltpu.SemaphoreType.DMA((2,)),
          pltpu.SemaphoreType.DMA((4,)),
      ],
  )(dst)


def _sc_aggR(src, dst, table, feat):
  def body(src_hbm, dst_hbm, tab_hbm, out_hbm,
           sidx, didx, s16_v, d16_v, rows, rows16_v, acc, gsem, ssem, isem):
    c = lax.axis_index("c")
    s = lax.axis_index("s")
    wid = s * NC + c

    def zr(i, _):
      for k in range(feat // 16):
        rows[0, i, pl.ds(k * 16, 16)] = jnp.zeros((16,), _f32)
      return 0
    lax.fori_loop(0, EBLK, zr, 0)
    for j in range(RPS // EBLK):
      pltpu.sync_copy(rows.at[0], acc.at[pl.ds(s * RPS + j * EBLK, EBLK)])
    plsc.subcore_barrier()

    base = wid * EPW

    def start_idx(g):
      gi = lax.rem(g, 4)
      pltpu.async_copy(src_hbm.at[pl.ds(base + g * EBLK, EBLK)], sidx.at[gi],
                       isem.at[gi])
      pltpu.async_copy(dst_hbm.at[pl.ds(base + g * EBLK, EBLK)], didx.at[gi],
                       isem.at[gi])

    start_idx(0)
    start_idx(1)

    def it(g, _):
      si = lax.rem(g, 4)
      sr = lax.rem(g, 2)

      @pl.when(g >= 2)
      def _():
        pltpu.make_async_copy(tab_hbm.at[pl.ds(0, EBLK)], rows.at[sr],
                              ssem.at[sr]).wait()

      @pl.when(g + 2 < NFULL)
      def _():
        start_idx(g + 2)

      pltpu.make_async_copy(src_hbm.at[pl.ds(base, EBLK)], sidx.at[si],
                            isem.at[si]).wait()
      pltpu.make_async_copy(src_hbm.at[pl.ds(base, EBLK)], didx.at[si],
                            isem.at[si]).wait()
      pltpu.async_copy(tab_hbm.at[sidx.at[si]], rows.at[sr], gsem).wait()
      pltpu.async_copy(rows.at[sr], acc.at[didx.at[si]], ssem.at[sr], add=True)
      return 0
    lax.fori_loop(0, NFULL, it, 0)
    pltpu.make_async_copy(tab_hbm.at[pl.ds(0, EBLK)], rows.at[0], ssem.at[0]).wait()
    pltpu.make_async_copy(tab_hbm.at[pl.ds(0, EBLK)], rows.at[1], ssem.at[1]).wait()

    pltpu.sync_copy(src_hbm.at[pl.ds(base + NFULL * EBLK, ETAIL)], s16_v)
    pltpu.sync_copy(dst_hbm.at[pl.ds(base + NFULL * EBLK, ETAIL)], d16_v)
    pltpu.async_copy(tab_hbm.at[s16_v], rows16_v, gsem).wait()
    pltpu.sync_copy(rows16_v, acc.at[d16_v], add=True)

    plsc.subcore_barrier()
    for j in range(RPS // EBLK):
      pltpu.sync_copy(acc.at[pl.ds(s * RPS + j * EBLK, EBLK)], rows.at[0])
      pltpu.sync_copy(rows.at[0], out_hbm.at[pl.ds(c * NP + s * RPS + j * EBLK, EBLK)])

  return pl.kernel(
      body,
      out_type=jax.ShapeDtypeStruct((NC * NP, feat), _f32),
      mesh=_sc_mesh(),
      compiler_params=pltpu.CompilerParams(use_tc_tiling_on_sc=(feat == F)),
      scratch_types=[
          pltpu.VMEM((4, EBLK), _i32),
          pltpu.VMEM((4, EBLK), _i32),
          pltpu.VMEM((ETAIL,), _i32),
          pltpu.VMEM((ETAIL,), _i32),
          pltpu.VMEM((2, EBLK, feat), _f32),
          pltpu.VMEM((ETAIL, feat), _f32),
          pltpu.VMEM_SHARED((NP, feat), _f32),
          pltpu.SemaphoreType.DMA,
          pltpu.SemaphoreType.DMA((2,)),
          pltpu.SemaphoreType.DMA((4,)),
      ],
  )(src, dst, table)


def _lstm_body(seq_ref, len_ref, emb_ref, wihf_ref, whhf_ref, bf_ref,
               wihb_ref, whhb_ref, bb_ref, out_ref):
  ef = lax.dot_general(emb_ref[...], wihf_ref[...], (((1,), (1,)), ((), ())),
                       preferred_element_type=_f32)
  eb = lax.dot_general(emb_ref[...], wihb_ref[...], (((1,), (1,)), ((), ())),
                       preferred_element_type=_f32)
  z21 = jnp.zeros((21, 4 * H), _f32)
  ecat = jnp.concatenate(
      [jnp.concatenate([ef, z21], 1), jnp.concatenate([z21, eb], 1)], 0)
  zw = jnp.zeros((4 * H, H), _f32)
  wcat = jnp.concatenate(
      [jnp.concatenate([whhf_ref[...], zw], 1),
       jnp.concatenate([zw, whhb_ref[...]], 1)], 0)
  bcat = jnp.concatenate([bf_ref[...], bb_ref[...]], 1)
  lens = len_ref[...]

  def onehot_t(t):
    row = seq_ref[t, :]
    return (lax.broadcasted_iota(_i32, (21, B), 0) == row[None, :]).astype(_f32)

  def step(t, carry):
    hf, cf, hb, cb = carry
    sp = (T - 1) - t
    a = jnp.concatenate([onehot_t(t), onehot_t(sp)], 0)
    hcat = jnp.concatenate([hf, hb], 1)
    g = (lax.dot_general(a, ecat, (((0,), (0,)), ((), ())),
                         preferred_element_type=_f32)
         + lax.dot_general(hcat, wcat, (((1,), (1,)), ((), ())),
                           preferred_element_type=_f32) + bcat)
    sg = jax.nn.sigmoid(g)
    th = jnp.tanh(g)
    cnf = sg[:, H:2 * H] * cf + sg[:, 0:H] * th[:, 2 * H:3 * H]
    cnb = sg[:, 5 * H:6 * H] * cb + sg[:, 4 * H:5 * H] * th[:, 6 * H:7 * H]
    tc2 = jnp.tanh(jnp.concatenate([cnf, cnb], 1))
    hnf = sg[:, 3 * H:4 * H] * tc2[:, 0:H]
    hnb = sg[:, 7 * H:8 * H] * tc2[:, H:2 * H]
    mf = t < lens
    mb = sp < lens
    hf = jnp.where(mf, hnf, hf)
    cf = jnp.where(mf, cnf, cf)
    hb = jnp.where(mb, hnb, hb)
    cb = jnp.where(mb, cnb, cb)
    return hf, cf, hb, cb

  z = jnp.zeros((B, H), _f32)
  hf, _, hb, _ = lax.fori_loop(0, T, step, (z, z, z, z))
  out_ref[:, 0:H] = hf
  out_ref[:, H:2 * H] = hb


def _lstm(seq_tm, lens_col, emb, wihf, whhf, bf, wihb, whhb, bb):
  return pl.pallas_call(
      _lstm_body,
      out_shape=jax.ShapeDtypeStruct((B, 2 * H), _f32),
  )(seq_tm, lens_col, emb, wihf, whhf, bf, wihb, whhb, bb)


_NB = 2048
_NG = NP // _NB


def _tab(i):
  return pl.BlockSpec((_NB, FD), lambda i: (i, 0))


def _degtab_body(p0, p1, out):
  out[...] = p0[...] + p1[...]


def _degtab(p):
  spec0 = pl.BlockSpec((_NB, FD), lambda i: (i, 0))
  spec1 = pl.BlockSpec((_NB, FD), lambda i: (i + _NG, 0))
  return pl.pallas_call(
      _degtab_body,
      grid=(_NG,),
      in_specs=[spec0, spec1],
      out_specs=pl.BlockSpec((_NB, FD), lambda i: (i, 0)),
      out_shape=jax.ShapeDtypeStruct((NP, FD), _f32),
  )(p, p)


def _h1_body(dt, q0, q1, w1, b1, out):
  d = dt[...][:, 0:1]
  q = q0[...][:, 0:1] + q1[...][:, 0:1]
  a = jnp.where(d > 0, q / jnp.maximum(d, 1.0), d)
  out[...] = jnp.maximum(a * w1[...] + b1[...], 0.0)


def _h1(dtab, qp, w1row, b1row):
  tab0 = pl.BlockSpec((_NB, FD), lambda i: (i, 0))
  tab1 = pl.BlockSpec((_NB, FD), lambda i: (i + _NG, 0))
  return pl.pallas_call(
      _h1_body,
      grid=(_NG,),
      in_specs=[tab0, tab0, tab1,
                pl.BlockSpec((1, F), lambda i: (0, 0)),
                pl.BlockSpec((1, F), lambda i: (0, 0))],
      out_specs=pl.BlockSpec((_NB, F), lambda i: (i, 0)),
      out_shape=jax.ShapeDtypeStruct((NP, F), _f32),
  )(dtab, qp, qp, w1row, b1row)


def _h2g3_body(p0, p1, h1, dt, w2, b2, w3, out):
  d = dt[...][:, 0:1]
  a2 = jnp.where(d > 0, (p0[...] + p1[...]) / jnp.maximum(d, 1.0), h1[...])
  h2 = jnp.maximum(
      lax.dot_general(a2, w2[...], (((1,), (1,)), ((), ())),
                      preferred_element_type=_f32) + b2[...], 0.0)
  out[...] = lax.dot_general(h2, w3[...], (((1,), (1,)), ((), ())),
                             preferred_element_type=_f32)


def _h2g3(p, h1, dtab, w2, b2, w3):
  row0 = pl.BlockSpec((_NB, F), lambda i: (i, 0))
  row1 = pl.BlockSpec((_NB, F), lambda i: (i + _NG, 0))
  tab0 = pl.BlockSpec((_NB, FD), lambda i: (i, 0))
  return pl.pallas_call(
      _h2g3_body,
      grid=(_NG,),
      in_specs=[row0, row1, row0, tab0,
                pl.BlockSpec((256, F), lambda i: (0, 0)),
                pl.BlockSpec((1, 256), lambda i: (0, 0)),
                pl.BlockSpec((F, 256), lambda i: (0, 0))],
      out_specs=row0,
      out_shape=jax.ShapeDtypeStruct((NP, F), _f32),
  )(p, p, h1, dtab, w2, b2, w3)


def _final_body(p0, p1, g3, dt, gid, b3, co, wrl, wrg, br, out,
                gsum_sc, gcnt_sc):
  i = pl.program_id(0)
  nblk = p0.shape[0]
  d = dt[...][:, 0:1]
  a3 = jnp.where(d > 0, (p0[...] + p1[...]) / jnp.maximum(d, 1.0), g3[...])
  h3 = jnp.maximum(a3 + b3[...], 0.0)
  oh = (gid[...] == lax.broadcasted_iota(_i32, (nblk, NGRAPH), 1)).astype(_f32)
  part = lax.dot_general(oh, h3, (((0,), (0,)), ((), ())),
                         preferred_element_type=_f32)
  cnt = lax.dot_general(oh, jnp.ones((nblk, 1), _f32), (((0,), (0,)), ((), ())),
                        preferred_element_type=_f32)

  @pl.when(i == 0)
  def _():
    gsum_sc[...] = jnp.zeros_like(gsum_sc)
    gcnt_sc[...] = jnp.zeros_like(gcnt_sc)

  gsum_sc[...] += part
  gcnt_sc[...] += cnt

  @pl.when(i == pl.num_programs(0) - 1)
  def _():
    gmean = gsum_sc[...] / jnp.maximum(gcnt_sc[...], 1.0)
    out[...] = (lax.dot_general(co[...], wrl[...], (((1,), (1,)), ((), ())),
                                preferred_element_type=_f32)
                + lax.dot_general(gmean, wrg[...], (((1,), (1,)), ((), ())),
                                  preferred_element_type=_f32)
                + br[...])


def _final(p, g3, dtab, gid_col, b3row, co, wrl, wrg, br):
  row0 = pl.BlockSpec((_NB, F), lambda i: (i, 0))
  row1 = pl.BlockSpec((_NB, F), lambda i: (i + _NG, 0))
  tab0 = pl.BlockSpec((_NB, FD), lambda i: (i, 0))
  col0 = pl.BlockSpec((_NB, 1), lambda i: (i, 0))
  fixed = lambda shape: pl.BlockSpec(shape, lambda i: (0, 0))
  return pl.pallas_call(
      _final_body,
      grid=(_NG,),
      in_specs=[row0, row1, row0, tab0, col0,
                fixed((1, F)), fixed((B, 2 * H)), fixed((1, 2 * H)),
                fixed((1, F)), fixed((1, 1))],
      out_specs=fixed((B, 1)),
      out_shape=jax.ShapeDtypeStruct((B, 1), _f32),
      scratch_shapes=[pltpu.VMEM((NGRAPH, F), _f32),
                      pltpu.VMEM((NGRAPH, 1), _f32)],
  )(p, p, g3, dtab, gid_col, b3row, co, wrl, wrg, br)


def kernel(seq, seq_len, edge_index, graph_ids, emb, Wih_f, Whh_f, b_f,
           Wih_b, Whh_b, b_b, W1, b1, W2, b2, W3, b3, Wr, br):
  src = edge_index[0]
  dst = edge_index[1]

  seq_tm = seq.astype(_i32).T
  lens_col = seq_len.astype(_i32).reshape(B, 1)
  co = _lstm(seq_tm, lens_col, emb.astype(_f32), Wih_f, Whh_f,
             b_f.reshape(1, 4 * H), Wih_b, Whh_b, b_b.reshape(1, 4 * H))

  dp = _sc_degrows(dst)
  dtab = _degtab(dp)
  qp = _sc_aggR(src, dst, dtab, FD)
  h1 = _h1(dtab, qp, W1.reshape(1, F), b1.reshape(1, F))
  p2 = _sc_aggR(src, dst, h1, F)
  g3 = _h2g3(p2, h1, dtab, W2, b2.reshape(1, 256), W3)
  p3 = _sc_aggR(src, dst, g3, F)

  gid_pad = jnp.concatenate(
      [graph_ids.astype(_i32), jnp.full((NP - N_NODES,), NGRAPH, _i32)])
  out = _final(p3, g3, dtab, gid_pad.reshape(NP, 1), b3.reshape(1, F),
               co, Wr[:, :2 * H], Wr[:, 2 * H:], br.reshape(1, 1))
  return out
